# Initial kernel scaffold; baseline (speedup 1.0000x reference)
import jax
import jax.numpy as jnp
from jax.experimental import pallas as pl


def kernel(phoneme_indices, attention_scores, pratyahara_matrix, bias_scale):
    raise NotImplementedError("write your pallas kernel here")


# speedup vs baseline: 281.5776x; 281.5776x over previous
import functools

import jax
import jax.numpy as jnp
from jax import lax
from jax.experimental import pallas as pl
from jax.experimental.pallas import tpu as pltpu
from jax.experimental.pallas import tpu_sc as plsc

S = 2048
H = 12
VOCAB = 1024
TI = 256
TJ = 256


def _make_row_gather():
    info = plsc.get_sparse_core_info()
    nw = info.num_cores * info.num_subcores
    rows_per_w = S // nw

    mesh = plsc.VectorSubcoreMesh(core_axis_name="c", subcore_axis_name="s")

    @functools.partial(
        pl.kernel,
        mesh=mesh,
        out_type=jax.ShapeDtypeStruct((S, VOCAB), jnp.float32),
        scratch_types=[
            pltpu.VMEM((rows_per_w,), jnp.int32),
            pltpu.VMEM((rows_per_w, VOCAB), jnp.float32),
            pltpu.SemaphoreType.DMA,
        ],
    )
    def row_gather(table_hbm, idx_hbm, out_hbm, idx_v, rows_v, sem):
        wid = lax.axis_index("s") * info.num_cores + lax.axis_index("c")
        base = wid * rows_per_w
        pltpu.sync_copy(idx_hbm.at[pl.ds(base, rows_per_w)], idx_v)
        pltpu.async_copy(table_hbm.at[idx_v], rows_v, sem).wait()
        pltpu.sync_copy(rows_v, out_hbm.at[pl.ds(base, rows_per_w)])

    return row_gather


def _bias_add_body(idxj_ref, scale_ref, g_ref, scores_ref, out_ref):
    j_idx = idxj_ref[0, 0, :]
    iota = lax.broadcasted_iota(jnp.int32, (VOCAB, TJ), 0)
    onehot = (iota == j_idx[None, :]).astype(jnp.float32)
    rel = jnp.dot(
        g_ref[...], onehot,
        preferred_element_type=jnp.float32,
        precision=lax.Precision.HIGHEST,
    )
    scale = scale_ref[0, :]
    out_ref[...] = scores_ref[...] + scale[:, None, None] * rel[None, :, :]


def kernel(phoneme_indices, attention_scores, pratyahara_matrix, bias_scale):
    idx = jnp.clip(phoneme_indices.reshape(-1).astype(jnp.int32), 0, VOCAB - 1)

    gathered = _make_row_gather()(pratyahara_matrix, idx)

    idx_j = idx.reshape(S // TJ, 1, TJ)
    scores = attention_scores.reshape(H, S, S)
    scale = bias_scale.reshape(1, H)

    out = pl.pallas_call(
        _bias_add_body,
        grid=(S // TI, S // TJ),
        in_specs=[
            pl.BlockSpec((1, 1, TJ), lambda i, j: (j, 0, 0)),
            pl.BlockSpec((1, H), lambda i, j: (0, 0)),
            pl.BlockSpec((TI, VOCAB), lambda i, j: (i, 0)),
            pl.BlockSpec((H, TI, TJ), lambda i, j: (0, i, j)),
        ],
        out_specs=pl.BlockSpec((H, TI, TJ), lambda i, j: (0, i, j)),
        out_shape=jax.ShapeDtypeStruct((H, S, S), jnp.float32),
        compiler_params=pltpu.CompilerParams(
            dimension_semantics=("arbitrary", "arbitrary"),
        ),
    )(idx_j, scale, gathered, scores)

    return out.reshape(1, H, S, S)


# speedup vs baseline: 313.3537x; 1.1129x over previous
import functools

import jax
import jax.numpy as jnp
from jax import lax
from jax.experimental import pallas as pl
from jax.experimental.pallas import tpu as pltpu
from jax.experimental.pallas import tpu_sc as plsc

S = 2048
H = 12
VOCAB = 1024
TI = 512
TJ = 256


def _make_row_gather():
    info = plsc.get_sparse_core_info()
    nw = info.num_cores * info.num_subcores
    rows_per_w = S // nw

    mesh = plsc.VectorSubcoreMesh(core_axis_name="c", subcore_axis_name="s")

    @functools.partial(
        pl.kernel,
        mesh=mesh,
        out_type=jax.ShapeDtypeStruct((S, VOCAB), jnp.float32),
        scratch_types=[
            pltpu.VMEM((rows_per_w,), jnp.int32),
            pltpu.VMEM((rows_per_w, VOCAB), jnp.float32),
            pltpu.SemaphoreType.DMA,
        ],
    )
    def row_gather(table_hbm, idx_hbm, out_hbm, idx_v, rows_v, sem):
        wid = lax.axis_index("s") * info.num_cores + lax.axis_index("c")
        base = wid * rows_per_w
        pltpu.sync_copy(idx_hbm.at[pl.ds(base, rows_per_w)], idx_v)
        pltpu.async_copy(table_hbm.at[idx_v], rows_v, sem).wait()
        pltpu.sync_copy(rows_v, out_hbm.at[pl.ds(base, rows_per_w)])

    return row_gather


def _bias_add_body(idxj_ref, scale_ref, g_ref, scores_ref, out_ref):
    j_idx = idxj_ref[0, 0, :]
    iota = lax.broadcasted_iota(jnp.int32, (VOCAB, TJ), 0)
    onehot = (iota == j_idx[None, :]).astype(jnp.float32)
    rel = jnp.dot(
        g_ref[...], onehot,
        preferred_element_type=jnp.float32,
        precision=lax.Precision.HIGHEST,
    )
    scale = scale_ref[0, :]
    out_ref[...] = scores_ref[...] + scale[:, None, None] * rel[None, :, :]


def kernel(phoneme_indices, attention_scores, pratyahara_matrix, bias_scale):
    idx = jnp.clip(phoneme_indices.reshape(-1).astype(jnp.int32), 0, VOCAB - 1)

    gathered = _make_row_gather()(pratyahara_matrix, idx)

    idx_j = idx.reshape(S // TJ, 1, TJ)
    scores = attention_scores.reshape(H, S, S)
    scale = bias_scale.reshape(1, H)

    out = pl.pallas_call(
        _bias_add_body,
        grid=(S // TI, S // TJ),
        in_specs=[
            pl.BlockSpec((1, 1, TJ), lambda i, j: (j, 0, 0)),
            pl.BlockSpec((1, H), lambda i, j: (0, 0)),
            pl.BlockSpec((TI, VOCAB), lambda i, j: (i, 0)),
            pl.BlockSpec((H, TI, TJ), lambda i, j: (0, i, j)),
        ],
        out_specs=pl.BlockSpec((H, TI, TJ), lambda i, j: (0, i, j)),
        out_shape=jax.ShapeDtypeStruct((H, S, S), jnp.float32),
        compiler_params=pltpu.CompilerParams(
            dimension_semantics=("arbitrary", "arbitrary"),
        ),
    )(idx_j, scale, gathered, scores)

    return out.reshape(1, H, S, S)


# speedup vs baseline: 321.0727x; 1.0246x over previous
import functools

import jax
import jax.numpy as jnp
from jax import lax
from jax.experimental import pallas as pl
from jax.experimental.pallas import tpu as pltpu
from jax.experimental.pallas import tpu_sc as plsc

S = 2048
H = 12
VOCAB = 1024
TI = 512
TJ = 512


def _make_row_gather():
    info = plsc.get_sparse_core_info()
    nw = info.num_cores * info.num_subcores
    rows_per_w = S // nw

    mesh = plsc.VectorSubcoreMesh(core_axis_name="c", subcore_axis_name="s")

    @functools.partial(
        pl.kernel,
        mesh=mesh,
        out_type=jax.ShapeDtypeStruct((S, VOCAB), jnp.float32),
        scratch_types=[
            pltpu.VMEM((rows_per_w,), jnp.int32),
            pltpu.VMEM((rows_per_w, VOCAB), jnp.float32),
            pltpu.SemaphoreType.DMA,
        ],
    )
    def row_gather(table_hbm, idx_hbm, out_hbm, idx_v, rows_v, sem):
        wid = lax.axis_index("s") * info.num_cores + lax.axis_index("c")
        base = wid * rows_per_w
        pltpu.sync_copy(idx_hbm.at[pl.ds(base, rows_per_w)], idx_v)
        pltpu.async_copy(table_hbm.at[idx_v], rows_v, sem).wait()
        pltpu.sync_copy(rows_v, out_hbm.at[pl.ds(base, rows_per_w)])

    return row_gather


def _bias_add_body(idxj_ref, scale_ref, g_ref, scores_ref, out_ref):
    j_idx = idxj_ref[0, 0, :]
    iota = lax.broadcasted_iota(jnp.int32, (VOCAB, TJ), 0)
    onehot = (iota == j_idx[None, :]).astype(jnp.float32)
    rel = jnp.dot(
        g_ref[...], onehot,
        preferred_element_type=jnp.float32,
        precision=lax.Precision.HIGHEST,
    )
    scale = scale_ref[0, :]
    out_ref[...] = scores_ref[...] + scale[:, None, None] * rel[None, :, :]


def kernel(phoneme_indices, attention_scores, pratyahara_matrix, bias_scale):
    idx = jnp.clip(phoneme_indices.reshape(-1).astype(jnp.int32), 0, VOCAB - 1)

    gathered = _make_row_gather()(pratyahara_matrix, idx)

    idx_j = idx.reshape(S // TJ, 1, TJ)
    scores = attention_scores.reshape(H, S, S)
    scale = bias_scale.reshape(1, H)

    out = pl.pallas_call(
        _bias_add_body,
        grid=(S // TI, S // TJ),
        in_specs=[
            pl.BlockSpec((1, 1, TJ), lambda i, j: (j, 0, 0)),
            pl.BlockSpec((1, H), lambda i, j: (0, 0)),
            pl.BlockSpec((TI, VOCAB), lambda i, j: (i, 0)),
            pl.BlockSpec((H, TI, TJ), lambda i, j: (0, i, j)),
        ],
        out_specs=pl.BlockSpec((H, TI, TJ), lambda i, j: (0, i, j)),
        out_shape=jax.ShapeDtypeStruct((H, S, S), jnp.float32),
        compiler_params=pltpu.CompilerParams(
            dimension_semantics=("arbitrary", "arbitrary"),
        ),
    )(idx_j, scale, gathered, scores)

    return out.reshape(1, H, S, S)


# speedup vs baseline: 331.2814x; 1.0318x over previous
import functools

import jax
import jax.numpy as jnp
from jax import lax
from jax.experimental import pallas as pl
from jax.experimental.pallas import tpu as pltpu
from jax.experimental.pallas import tpu_sc as plsc

S = 2048
H = 12
VOCAB = 1024
TI = 512
TJ = 512


def _make_row_gather():
    info = plsc.get_sparse_core_info()
    nw = info.num_cores * info.num_subcores
    rows_per_w = S // nw

    mesh = plsc.VectorSubcoreMesh(core_axis_name="c", subcore_axis_name="s")

    @functools.partial(
        pl.kernel,
        mesh=mesh,
        out_type=jax.ShapeDtypeStruct((S, VOCAB), jnp.float32),
        scratch_types=[
            pltpu.VMEM((rows_per_w,), jnp.int32),
            pltpu.VMEM((rows_per_w, VOCAB), jnp.float32),
            pltpu.SemaphoreType.DMA,
        ],
    )
    def row_gather(table_hbm, idx_hbm, out_hbm, idx_v, rows_v, sem):
        wid = lax.axis_index("s") * info.num_cores + lax.axis_index("c")
        base = wid * rows_per_w
        pltpu.sync_copy(idx_hbm.at[pl.ds(base, rows_per_w)], idx_v)
        pltpu.async_copy(table_hbm.at[idx_v], rows_v, sem).wait()
        pltpu.sync_copy(rows_v, out_hbm.at[pl.ds(base, rows_per_w)])

    return row_gather


def _bias_add_body(idxj_ref, scale_ref, g_ref, scores_ref, out_ref):
    j_idx = idxj_ref[0, 0, :]
    iota = lax.broadcasted_iota(jnp.int32, (VOCAB, TJ), 0)
    onehot = (iota == j_idx[None, :]).astype(jnp.float32)
    rel = jnp.dot(
        g_ref[...], onehot,
        preferred_element_type=jnp.float32,
        precision=lax.Precision.DEFAULT,
    )
    scale = scale_ref[0, :]
    out_ref[...] = scores_ref[...] + scale[:, None, None] * rel[None, :, :]


def kernel(phoneme_indices, attention_scores, pratyahara_matrix, bias_scale):
    idx = jnp.clip(phoneme_indices.reshape(-1).astype(jnp.int32), 0, VOCAB - 1)

    gathered = _make_row_gather()(pratyahara_matrix, idx)

    idx_j = idx.reshape(S // TJ, 1, TJ)
    scores = attention_scores.reshape(H, S, S)
    scale = bias_scale.reshape(1, H)

    out = pl.pallas_call(
        _bias_add_body,
        grid=(S // TI, S // TJ),
        in_specs=[
            pl.BlockSpec((1, 1, TJ), lambda i, j: (j, 0, 0)),
            pl.BlockSpec((1, H), lambda i, j: (0, 0)),
            pl.BlockSpec((TI, VOCAB), lambda i, j: (i, 0)),
            pl.BlockSpec((H, TI, TJ), lambda i, j: (0, i, j)),
        ],
        out_specs=pl.BlockSpec((H, TI, TJ), lambda i, j: (0, i, j)),
        out_shape=jax.ShapeDtypeStruct((H, S, S), jnp.float32),
        compiler_params=pltpu.CompilerParams(
            dimension_semantics=("arbitrary", "arbitrary"),
        ),
    )(idx_j, scale, gathered, scores)

    return out.reshape(1, H, S, S)
